# baseline (speedup 1.0000x reference)
import jax
import jax.numpy as jnp
import numpy as np
from jax.experimental import pallas as pl
from jax.experimental.pallas import tpu as pltpu

try:
    from jax import shard_map
except ImportError:
    from jax.experimental.shard_map import shard_map

_N = 9216
_C = 196
_D = _C * 16
_E = 64
_D12 = _C * 5
_BLK = 512


def _router_kernel(thr_ref, temp_ref, keys_ref, x_ref, o_ref, keff_ref, w_ref):
    @pl.when(pl.program_id(0) == 0)
    def _build_constants():
        q = jax.lax.broadcasted_iota(jnp.int32, (_D12, _D), 0)
        p = jax.lax.broadcasted_iota(jnp.int32, (_D12, _D), 1)
        c = p >> 4
        i = (p >> 3) & 1
        j = (p >> 1) & 1
        idx2 = _C + (c << 2) + (i << 1) + j
        expand = (jnp.where(q == c, 1.0 / 16.0, 0.0)
                  + jnp.where(q == idx2, 0.25, 0.0))
        k12 = keys_ref[:, 0:_D12]
        keff = keys_ref[:, _D12:] + jax.lax.dot_general(
            k12, expand, (((1,), (0,)), ((), ())),
            preferred_element_type=jnp.float32)
        keff_ref[...] = keff.astype(jnp.bfloat16)

        r = jax.lax.broadcasted_iota(jnp.int32, (_D, 4), 0)
        col = jax.lax.broadcasted_iota(jnp.int32, (_D, 4), 1)
        s = r & 15
        w = (jnp.where(col == 0, 1.0, 0.0)
             + jnp.where((col == 1) & ((s & 5) == 0), 1.0 / 16.0, 0.0)
             + jnp.where((col == 2) & (s == 0), 1.0 / 256.0, 0.0))
        w_ref[...] = w.astype(jnp.bfloat16)

    x = x_ref[...]
    xb = x.astype(jnp.bfloat16)

    def rot(a, k):
        return jnp.concatenate([a[:, k:], a[:, :k]], axis=1)

    y = xb + rot(xb, 1)
    z = y + rot(y, 4)
    u = z + rot(z, 2)
    v = u + rot(u, 8)

    w = w_ref[...]
    dims = (((1,), (0,)), ((), ()))
    n4 = jax.lax.dot_general(xb * xb, w, dims,
                             preferred_element_type=jnp.float32)[:, 0:1]
    n2 = jax.lax.dot_general(z * z, w, dims,
                             preferred_element_type=jnp.float32)[:, 1:2]
    n1 = jax.lax.dot_general(v * v, w, dims,
                             preferred_element_type=jnp.float32)[:, 2:3]
    norm2 = n4 + n2 + n1

    logits = jax.lax.dot_general(
        xb, keff_ref[...], (((1,), (1,)), ((), ())),
        preferred_element_type=jnp.float32)

    inv = 1.0 / jnp.maximum(jnp.sqrt(norm2), 1e-12)
    l = logits * inv
    m = jnp.max(l, axis=1, keepdims=True)
    e = jnp.exp(l - m)
    se = jnp.sum(e, axis=1, keepdims=True)
    wgt = e / se
    max_w = 1.0 / se
    at = jnp.clip(thr_ref[0, 0] * (2.0 - max_w), 0.01, 0.8)
    mask = jax.nn.sigmoid(temp_ref[0, 0] * (wgt - at))
    wf = wgt * mask
    sw = jnp.sum(wf, axis=1, keepdims=True)
    o_ref[...] = wf / jnp.maximum(sw, 1e-8)


def _run(thr, temp, keys, xf):
    n = xf.shape[0]
    grid = (n // _BLK,)
    return pl.pallas_call(
        _router_kernel,
        grid=grid,
        in_specs=[
            pl.BlockSpec((1, 1), lambda i: (0, 0)),
            pl.BlockSpec((1, 1), lambda i: (0, 0)),
            pl.BlockSpec((_E, _D12 + _D), lambda i: (0, 0)),
            pl.BlockSpec((_BLK, _D), lambda i: (i, 0)),
        ],
        out_specs=pl.BlockSpec((_BLK, _E), lambda i: (i, 0)),
        out_shape=jax.ShapeDtypeStruct((n, _E), jnp.float32),
        scratch_shapes=[pltpu.VMEM((_E, _D), jnp.bfloat16),
                        pltpu.VMEM((_D, 4), jnp.bfloat16)],
        compiler_params=pltpu.CompilerParams(
            dimension_semantics=("arbitrary",)),
    )(thr, temp, keys, xf)


def kernel(patch, threshold, keys, temperature):
    n = patch.shape[0]
    xf = patch.reshape(n, _D)
    thr = jnp.reshape(threshold, (1, 1)).astype(jnp.float32)
    temp = jnp.reshape(temperature, (1, 1)).astype(jnp.float32)

    devs = jax.devices()
    n_shard = 1
    for cand in (2,):
        if len(devs) >= cand and n % (cand * _BLK) == 0:
            n_shard = cand
    if n_shard > 1:
        mesh = jax.sharding.Mesh(np.asarray(devs[:n_shard]), ("x",))
        spec = jax.sharding.PartitionSpec
        fn = shard_map(_run, mesh=mesh,
                       in_specs=(spec(None, None), spec(None, None),
                                 spec(None, None), spec("x", None)),
                       out_specs=spec("x", None), check_vma=False)
        return fn(thr, temp, keys, xf)
    return _run(thr, temp, keys, xf)


# speedup vs baseline: 2.0448x; 2.0448x over previous
import jax
import jax.numpy as jnp
from jax.experimental import pallas as pl
from jax.experimental.pallas import tpu as pltpu

_N = 9216
_C = 196
_D = _C * 16
_E = 64
_D12 = _C * 5
_BLK = 1024


def _router_kernel(thr_ref, temp_ref, keys_ref, x_ref, o_ref, keff_ref, w_ref):
    @pl.when(pl.program_id(0) == 0)
    def _build_constants():
        q = jax.lax.broadcasted_iota(jnp.int32, (_D12, _D), 0)
        p = jax.lax.broadcasted_iota(jnp.int32, (_D12, _D), 1)
        c = p >> 4
        i = (p >> 3) & 1
        j = (p >> 1) & 1
        idx2 = _C + (c << 2) + (i << 1) + j
        expand = (jnp.where(q == c, 1.0 / 16.0, 0.0)
                  + jnp.where(q == idx2, 0.25, 0.0))
        k12 = keys_ref[:, 0:_D12]
        keff = keys_ref[:, _D12:] + jax.lax.dot_general(
            k12, expand, (((1,), (0,)), ((), ())),
            preferred_element_type=jnp.float32)
        keff_ref[...] = keff.astype(jnp.bfloat16)

        r = jax.lax.broadcasted_iota(jnp.int32, (_D, 4), 0)
        col = jax.lax.broadcasted_iota(jnp.int32, (_D, 4), 1)
        s = r & 15
        w = (jnp.where(col == 0, 1.0, 0.0)
             + jnp.where((col == 1) & ((s & 5) == 0), 1.0 / 16.0, 0.0)
             + jnp.where((col == 2) & (s == 0), 1.0 / 256.0, 0.0))
        w_ref[...] = w.astype(jnp.bfloat16)

    x = x_ref[...]
    xb = x.astype(jnp.bfloat16)

    def rot(a, k):
        return jnp.concatenate([a[:, k:], a[:, :k]], axis=1)

    y = xb + rot(xb, 1)
    z = y + rot(y, 4)
    u = z + rot(z, 2)
    v = u + rot(u, 8)

    w = w_ref[...]
    dims = (((1,), (0,)), ((), ()))
    n4 = jax.lax.dot_general(xb * xb, w, dims,
                             preferred_element_type=jnp.float32)[:, 0:1]
    n2 = jax.lax.dot_general(z * z, w, dims,
                             preferred_element_type=jnp.float32)[:, 1:2]
    n1 = jax.lax.dot_general(v * v, w, dims,
                             preferred_element_type=jnp.float32)[:, 2:3]
    norm2 = n4 + n2 + n1

    logits = jax.lax.dot_general(
        xb, keff_ref[...], (((1,), (1,)), ((), ())),
        preferred_element_type=jnp.float32)

    inv = 1.0 / jnp.maximum(jnp.sqrt(norm2), 1e-12)
    l = logits * inv
    m = jnp.max(l, axis=1, keepdims=True)
    e = jnp.exp(l - m)
    se = jnp.sum(e, axis=1, keepdims=True)
    wgt = e / se
    max_w = 1.0 / se
    at = jnp.clip(thr_ref[0, 0] * (2.0 - max_w), 0.01, 0.8)
    mask = jax.nn.sigmoid(temp_ref[0, 0] * (wgt - at))
    wf = wgt * mask
    sw = jnp.sum(wf, axis=1, keepdims=True)
    o_ref[...] = wf / jnp.maximum(sw, 1e-8)


def kernel(patch, threshold, keys, temperature):
    n = patch.shape[0]
    xf = patch.reshape(n, _D)
    thr = jnp.reshape(threshold, (1, 1)).astype(jnp.float32)
    temp = jnp.reshape(temperature, (1, 1)).astype(jnp.float32)
    grid = (n // _BLK,)
    out = pl.pallas_call(
        _router_kernel,
        grid=grid,
        in_specs=[
            pl.BlockSpec((1, 1), lambda i: (0, 0)),
            pl.BlockSpec((1, 1), lambda i: (0, 0)),
            pl.BlockSpec((_E, _D12 + _D), lambda i: (0, 0)),
            pl.BlockSpec((_BLK, _D), lambda i: (i, 0)),
        ],
        out_specs=pl.BlockSpec((_BLK, _E), lambda i: (i, 0)),
        out_shape=jax.ShapeDtypeStruct((n, _E), jnp.float32),
        scratch_shapes=[pltpu.VMEM((_E, _D), jnp.bfloat16),
                        pltpu.VMEM((_D, 4), jnp.bfloat16)],
        compiler_params=pltpu.CompilerParams(
            dimension_semantics=("arbitrary",)),
    )(thr, temp, keys, xf)
    return out
